# Initial kernel scaffold; baseline (speedup 1.0000x reference)
import jax
import jax.numpy as jnp
from jax.experimental import pallas as pl


def kernel(input, weight):
    raise NotImplementedError("write your pallas kernel here")


# speedup vs baseline: 1.6147x; 1.6147x over previous
import jax
import jax.numpy as jnp
from jax import lax
from jax.experimental import pallas as pl
from jax.experimental.pallas import tpu as pltpu
from jax.experimental.pallas import tpu_sc as plsc

VOCAB = 100000
HIDDEN = 4096
TOKENS = 8192

NC = 2
NS = 16
NW = NC * NS
TOK_PER_W = TOKENS // NW
C = 16
NCHUNK = TOK_PER_W // C

_mesh = plsc.VectorSubcoreMesh(
    core_axis_name="c", subcore_axis_name="s", num_cores=NC, num_subcores=NS
)


@jax.jit
def _embed(weight, idx3):
    def body(table_hbm, idx_hbm, out_hbm, idx_v, buf, sem):
        wid = lax.axis_index("s") * NC + lax.axis_index("c")
        base = wid * TOK_PER_W
        pltpu.sync_copy(idx_hbm.at[wid], idx_v)

        @pl.loop(0, NCHUNK)
        def _(j):
            pltpu.async_copy(table_hbm.at[idx_v.at[j]], buf, sem).wait()
            pltpu.sync_copy(buf, out_hbm.at[pl.ds(base + j * C, C)])

    f = pl.kernel(
        body,
        out_type=jax.ShapeDtypeStruct((TOKENS, HIDDEN), jnp.float32),
        mesh=_mesh,
        scratch_types=[
            pltpu.VMEM((NCHUNK, C), jnp.int32),
            pltpu.VMEM((C, HIDDEN), jnp.float32),
            pltpu.SemaphoreType.DMA,
        ],
    )
    return f(weight, idx3)


def kernel(input, weight):
    idx3 = input.reshape(NW, NCHUNK, C)
    return _embed(weight, idx3)


# speedup vs baseline: 1.7620x; 1.0913x over previous
import jax
import jax.numpy as jnp
from jax import lax
from jax.experimental import pallas as pl
from jax.experimental.pallas import tpu as pltpu
from jax.experimental.pallas import tpu_sc as plsc

VOCAB = 100000
HIDDEN = 4096
TOKENS = 8192

NC = 2
NS = 16
NW = NC * NS
TOK_PER_W = TOKENS // NW
C = 8
NCHUNK = TOK_PER_W // C

_mesh = plsc.VectorSubcoreMesh(
    core_axis_name="c", subcore_axis_name="s", num_cores=NC, num_subcores=NS
)


@jax.jit
def _embed(weight, idx3):
    def body(table_hbm, idx_hbm, out_hbm, idx_v, buf0, buf1,
             gsem0, gsem1, ssem0, ssem1):
        wid = lax.axis_index("s") * NC + lax.axis_index("c")
        base = wid * TOK_PER_W
        pltpu.sync_copy(idx_hbm.at[wid], idx_v)

        bufs = (buf0, buf1)
        gsems = (gsem0, gsem1)
        ssems = (ssem0, ssem1)

        def gather_desc(j, b):
            return pltpu.make_async_copy(
                table_hbm.at[idx_v.at[j]], bufs[b], gsems[b])

        def scatter_desc(j, b):
            return pltpu.make_async_copy(
                bufs[b], out_hbm.at[pl.ds(base + j * C, C)], ssems[b])

        gather_desc(0, 0).start()

        @pl.loop(0, NCHUNK // 2)
        def _(g):
            j0 = 2 * g
            @pl.when(g > 0)
            def _():
                scatter_desc(j0 - 1, 1).wait()
            gather_desc(j0 + 1, 1).start()
            gather_desc(j0, 0).wait()
            scatter_desc(j0, 0).start()
            scatter_desc(j0, 0).wait()

            @pl.when(g < NCHUNK // 2 - 1)
            def _():
                gather_desc(j0 + 2, 0).start()
            gather_desc(j0 + 1, 1).wait()
            scatter_desc(j0 + 1, 1).start()

        scatter_desc(NCHUNK - 1, 1).wait()

    f = pl.kernel(
        body,
        out_type=jax.ShapeDtypeStruct((TOKENS, HIDDEN), jnp.float32),
        mesh=_mesh,
        scratch_types=[
            pltpu.VMEM((NCHUNK, C), jnp.int32),
            pltpu.VMEM((C, HIDDEN), jnp.float32),
            pltpu.VMEM((C, HIDDEN), jnp.float32),
            pltpu.SemaphoreType.DMA,
            pltpu.SemaphoreType.DMA,
            pltpu.SemaphoreType.DMA,
            pltpu.SemaphoreType.DMA,
        ],
    )
    return f(weight, idx3)


def kernel(input, weight):
    idx3 = input.reshape(NW, NCHUNK, C)
    return _embed(weight, idx3)
